# Initial kernel scaffold; baseline (speedup 1.0000x reference)
import jax
import jax.numpy as jnp
from jax.experimental import pallas as pl


def kernel(hidden_states, gate_weight):
    raise NotImplementedError("write your pallas kernel here")


# speedup vs baseline: 1.8010x; 1.8010x over previous
import jax
import jax.numpy as jnp
from jax.experimental import pallas as pl
from jax.experimental.pallas import tpu as pltpu

_N_EXPERTS = 8
_TOP_K = 2
_AUX_COEF = 0.01
_LANES = 128
_TILE = 2048


def _router_body(h_ref, w_ref, idx_ref, prob_ref, aux_ref, load_ref, cnt_ref):
    i = pl.program_id(0)
    n_total = pl.num_programs(0) * h_ref.shape[0]
    logits = jax.lax.dot_general(
        h_ref[...], w_ref[...],
        dimension_numbers=(((1,), (1,)), ((), ())),
        preferred_element_type=jnp.float32)
    lane = jax.lax.broadcasted_iota(jnp.int32, logits.shape, 1)
    valid = lane < _N_EXPERTS
    masked = jnp.where(valid, logits, jnp.float32(-1e30))
    m = jnp.max(masked, axis=1, keepdims=True)
    e = jnp.where(valid, jnp.exp(masked - m), jnp.float32(0.0))
    s = jnp.sum(e, axis=1, keepdims=True)
    probs = e / s
    pm = jnp.where(valid, probs, jnp.float32(-1.0))
    v1 = jnp.max(pm, axis=1, keepdims=True)
    i1 = jnp.min(jnp.where(pm == v1, lane, _LANES), axis=1, keepdims=True)
    pm2 = jnp.where(lane == i1, jnp.float32(-2.0), pm)
    v2 = jnp.max(pm2, axis=1, keepdims=True)
    i2 = jnp.min(jnp.where(pm2 == v2, lane, _LANES), axis=1, keepdims=True)
    idx_ref[...] = jnp.concatenate([i1, i2], axis=1)
    prob_ref[...] = jnp.concatenate([v1, v2], axis=1)
    hits = ((lane == i1) | (lane == i2)).astype(jnp.float32)
    part = jnp.sum(hits, axis=0, keepdims=True)

    @pl.when(i == 0)
    def _init():
        cnt_ref[...] = part

    @pl.when(i > 0)
    def _acc():
        cnt_ref[...] = cnt_ref[...] + part

    @pl.when(i == pl.num_programs(0) - 1)
    def _finalize():
        cnt = cnt_ref[...]
        load = cnt / jnp.float32(n_total * _TOP_K)
        lr = jax.lax.broadcasted_iota(jnp.int32, load.shape, 1)
        lvalid = lr < _N_EXPERTS
        mean = jnp.sum(jnp.where(lvalid, load, 0.0)) / _N_EXPERTS
        var = jnp.sum(jnp.where(lvalid, (load - mean) ** 2, 0.0)) / _N_EXPERTS
        cv_sq = var / (mean * mean + 1e-9)
        aux_ref[...] = jnp.full(aux_ref.shape, _AUX_COEF * cv_sq, jnp.float32)
        load_ref[...] = load


def kernel(hidden_states, gate_weight):
    n, d = hidden_states.shape
    tile = _TILE
    grid = n // tile
    wpad = jnp.pad(gate_weight, ((0, _LANES - _N_EXPERTS), (0, 0)))
    out_shapes = (
        jax.ShapeDtypeStruct((n, _TOP_K), jnp.int32),
        jax.ShapeDtypeStruct((n, _TOP_K), jnp.float32),
        jax.ShapeDtypeStruct((1, _LANES), jnp.float32),
        jax.ShapeDtypeStruct((1, _LANES), jnp.float32),
    )
    idx, prob, aux, loadp = pl.pallas_call(
        _router_body,
        grid=(grid,),
        in_specs=[
            pl.BlockSpec((tile, d), lambda i: (i, 0)),
            pl.BlockSpec((_LANES, d), lambda i: (0, 0)),
        ],
        out_specs=(
            pl.BlockSpec((tile, _TOP_K), lambda i: (i, 0)),
            pl.BlockSpec((tile, _TOP_K), lambda i: (i, 0)),
            pl.BlockSpec((1, _LANES), lambda i: (0, 0)),
            pl.BlockSpec((1, _LANES), lambda i: (0, 0)),
        ),
        out_shape=out_shapes,
        scratch_shapes=[pltpu.VMEM((1, _LANES), jnp.float32)],
    )(hidden_states, wpad)
    return (idx, prob, aux[0, 0], loadp[0, :_N_EXPERTS])


# speedup vs baseline: 1.8298x; 1.0160x over previous
import jax
import jax.numpy as jnp
from jax.experimental import pallas as pl
from jax.experimental.pallas import tpu as pltpu

_N_EXPERTS = 8
_TOP_K = 2
_AUX_COEF = 0.01
_LANES = 128
_TILE = 2048


def _router_body(h_ref, w_ref, b_ref, idx_ref, prob_ref, aux_ref, load_ref,
                 cnt_ref):
    i = pl.program_id(0)
    n_total = pl.num_programs(0) * h_ref.shape[0]
    logits = jax.lax.dot_general(
        h_ref[...], w_ref[...],
        dimension_numbers=(((1,), (1,)), ((), ())),
        preferred_element_type=jnp.float32)
    lane = jax.lax.broadcasted_iota(jnp.int32, logits.shape, 1)
    masked = logits + b_ref[...]
    m = jnp.max(masked, axis=1, keepdims=True)
    e = jnp.exp(masked - m)
    s = jnp.sum(e, axis=1, keepdims=True)
    i1 = jnp.min(jnp.where(masked == m, lane, _LANES), axis=1, keepdims=True)
    masked2 = jnp.where(lane == i1, jnp.float32(-2e30), masked)
    v2 = jnp.max(masked2, axis=1, keepdims=True)
    i2 = jnp.min(jnp.where(masked2 == v2, lane, _LANES), axis=1, keepdims=True)
    p1 = 1.0 / s
    p2 = jnp.exp(v2 - m) / s
    idx_ref[...] = jnp.concatenate([i1, i2], axis=1)
    prob_ref[...] = jnp.concatenate([p1, p2], axis=1)
    hits = jnp.where(masked >= v2, jnp.float32(1.0), jnp.float32(0.0))
    part = jnp.sum(hits, axis=0, keepdims=True)

    @pl.when(i == 0)
    def _init():
        cnt_ref[...] = part

    @pl.when(i > 0)
    def _acc():
        cnt_ref[...] = cnt_ref[...] + part

    @pl.when(i == pl.num_programs(0) - 1)
    def _finalize():
        cnt = cnt_ref[...]
        load = cnt / jnp.float32(n_total * _TOP_K)
        lr = jax.lax.broadcasted_iota(jnp.int32, load.shape, 1)
        lvalid = lr < _N_EXPERTS
        mean = jnp.sum(jnp.where(lvalid, load, 0.0)) / _N_EXPERTS
        var = jnp.sum(jnp.where(lvalid, (load - mean) ** 2, 0.0)) / _N_EXPERTS
        cv_sq = var / (mean * mean + 1e-9)
        aux_ref[...] = jnp.full(aux_ref.shape, _AUX_COEF * cv_sq, jnp.float32)
        load_ref[...] = load


def kernel(hidden_states, gate_weight):
    n, d = hidden_states.shape
    tile = _TILE
    grid = n // tile
    wpad = jnp.pad(gate_weight, ((0, _LANES - _N_EXPERTS), (0, 0)))
    bias = jnp.where(jnp.arange(_LANES) < _N_EXPERTS, 0.0,
                     -1e30).astype(jnp.float32).reshape(1, _LANES)
    out_shapes = (
        jax.ShapeDtypeStruct((n, _TOP_K), jnp.int32),
        jax.ShapeDtypeStruct((n, _TOP_K), jnp.float32),
        jax.ShapeDtypeStruct((1, _LANES), jnp.float32),
        jax.ShapeDtypeStruct((1, _LANES), jnp.float32),
    )
    idx, prob, aux, loadp = pl.pallas_call(
        _router_body,
        grid=(grid,),
        in_specs=[
            pl.BlockSpec((tile, d), lambda i: (i, 0)),
            pl.BlockSpec((_LANES, d), lambda i: (0, 0)),
            pl.BlockSpec((1, _LANES), lambda i: (0, 0)),
        ],
        out_specs=(
            pl.BlockSpec((tile, _TOP_K), lambda i: (i, 0)),
            pl.BlockSpec((tile, _TOP_K), lambda i: (i, 0)),
            pl.BlockSpec((1, _LANES), lambda i: (0, 0)),
            pl.BlockSpec((1, _LANES), lambda i: (0, 0)),
        ),
        out_shape=out_shapes,
        scratch_shapes=[pltpu.VMEM((1, _LANES), jnp.float32)],
    )(hidden_states, wpad, bias)
    return (idx, prob, aux[0, 0], loadp[0, :_N_EXPERTS])


# speedup vs baseline: 1.9418x; 1.0612x over previous
import jax
import jax.numpy as jnp
from jax.experimental import pallas as pl
from jax.experimental.pallas import tpu as pltpu

_N_EXPERTS = 8
_TOP_K = 2
_AUX_COEF = 0.01
_LANES = 128
_TILE = 4096


def _router_body(h_ref, w_ref, b_ref, idx_ref, prob_ref, aux_ref, load_ref,
                 cnt_ref):
    i = pl.program_id(0)
    n_total = pl.num_programs(0) * h_ref.shape[0]
    logits = jax.lax.dot_general(
        h_ref[...], w_ref[...],
        dimension_numbers=(((1,), (1,)), ((), ())),
        preferred_element_type=jnp.float32)
    lane = jax.lax.broadcasted_iota(jnp.int32, logits.shape, 1)
    masked = logits + b_ref[...]
    m = jnp.max(masked, axis=1, keepdims=True)
    e = jnp.exp(masked - m)
    s = jnp.sum(e, axis=1, keepdims=True)
    i1 = jnp.min(jnp.where(masked == m, lane, _LANES), axis=1, keepdims=True)
    masked2 = jnp.where(lane == i1, jnp.float32(-2e30), masked)
    v2 = jnp.max(masked2, axis=1, keepdims=True)
    i2 = jnp.min(jnp.where(masked2 == v2, lane, _LANES), axis=1, keepdims=True)
    p1 = 1.0 / s
    p2 = jnp.exp(v2 - m) / s
    idx_ref[...] = jnp.concatenate([i1, i2], axis=1)
    prob_ref[...] = jnp.concatenate([p1, p2], axis=1)
    hits = jnp.where(masked >= v2, jnp.float32(1.0), jnp.float32(0.0))
    part = jnp.sum(hits, axis=0, keepdims=True)

    @pl.when(i == 0)
    def _init():
        cnt_ref[...] = part

    @pl.when(i > 0)
    def _acc():
        cnt_ref[...] = cnt_ref[...] + part

    @pl.when(i == pl.num_programs(0) - 1)
    def _finalize():
        cnt = cnt_ref[...]
        load = cnt / jnp.float32(n_total * _TOP_K)
        lr = jax.lax.broadcasted_iota(jnp.int32, load.shape, 1)
        lvalid = lr < _N_EXPERTS
        mean = jnp.sum(jnp.where(lvalid, load, 0.0)) / _N_EXPERTS
        var = jnp.sum(jnp.where(lvalid, (load - mean) ** 2, 0.0)) / _N_EXPERTS
        cv_sq = var / (mean * mean + 1e-9)
        aux_ref[...] = jnp.full(aux_ref.shape, _AUX_COEF * cv_sq, jnp.float32)
        load_ref[...] = load


def kernel(hidden_states, gate_weight):
    n, d = hidden_states.shape
    tile = _TILE
    grid = n // tile
    wpad = jnp.pad(gate_weight, ((0, _LANES - _N_EXPERTS), (0, 0)))
    bias = jnp.where(jnp.arange(_LANES) < _N_EXPERTS, 0.0,
                     -1e30).astype(jnp.float32).reshape(1, _LANES)
    out_shapes = (
        jax.ShapeDtypeStruct((n, _TOP_K), jnp.int32),
        jax.ShapeDtypeStruct((n, _TOP_K), jnp.float32),
        jax.ShapeDtypeStruct((1, _LANES), jnp.float32),
        jax.ShapeDtypeStruct((1, _LANES), jnp.float32),
    )
    idx, prob, aux, loadp = pl.pallas_call(
        _router_body,
        grid=(grid,),
        in_specs=[
            pl.BlockSpec((tile, d), lambda i: (i, 0)),
            pl.BlockSpec((_LANES, d), lambda i: (0, 0)),
            pl.BlockSpec((1, _LANES), lambda i: (0, 0)),
        ],
        out_specs=(
            pl.BlockSpec((tile, _TOP_K), lambda i: (i, 0)),
            pl.BlockSpec((tile, _TOP_K), lambda i: (i, 0)),
            pl.BlockSpec((1, _LANES), lambda i: (0, 0)),
            pl.BlockSpec((1, _LANES), lambda i: (0, 0)),
        ),
        out_shape=out_shapes,
        scratch_shapes=[pltpu.VMEM((1, _LANES), jnp.float32)],
    )(hidden_states, wpad, bias)
    return (idx, prob, aux[0, 0], loadp[0, :_N_EXPERTS])


# speedup vs baseline: 2.0788x; 1.0705x over previous
import functools

import jax
import jax.numpy as jnp
from jax import lax
from jax.experimental import pallas as pl
from jax.experimental.pallas import tpu as pltpu
from jax.experimental.pallas import tpu_sc as plsc

_N_EXPERTS = 8
_TOP_K = 2
_AUX_COEF = 0.01
_LANES = 128
_TILE = 4096
_NW = 32
_L = 16


def _logits_body(h_ref, w_ref, o_ref):
    logits_t = jax.lax.dot_general(
        w_ref[...], h_ref[...],
        dimension_numbers=(((1,), (1,)), ((), ())),
        preferred_element_type=jnp.float32)
    o_ref[...] = logits_t[:_N_EXPERTS, :]


def _stats_body(pc_ref, aux_ref, load_ref, n_total):
    rows = jnp.sum(pc_ref[...], axis=0, keepdims=True)
    lr = jax.lax.broadcasted_iota(jnp.int32, rows.shape, 1)
    cnt = jnp.zeros_like(rows)
    for e in range(_N_EXPERTS):
        blk = jnp.logical_and(lr >= e * _L, lr < (e + 1) * _L)
        tot = jnp.sum(jnp.where(blk, rows, 0.0))
        cnt = cnt + jnp.where(lr == e, tot, 0.0)
    load = cnt / jnp.float32(n_total * _TOP_K)
    lvalid = lr < _N_EXPERTS
    mean = jnp.sum(jnp.where(lvalid, load, 0.0)) / _N_EXPERTS
    var = jnp.sum(jnp.where(lvalid, (load - mean) ** 2, 0.0)) / _N_EXPERTS
    cv_sq = var / (mean * mean + 1e-9)
    aux_ref[...] = jnp.full(aux_ref.shape, _AUX_COEF * cv_sq, jnp.float32)
    load_ref[...] = load


def _sc_router(logits_flat, n):
    chunk = n // _NW
    groups = chunk // _L
    mesh = plsc.VectorSubcoreMesh(core_axis_name="c", subcore_axis_name="s")

    @functools.partial(
        pl.kernel,
        out_type=(
            jax.ShapeDtypeStruct((n,), jnp.int32),
            jax.ShapeDtypeStruct((n,), jnp.int32),
            jax.ShapeDtypeStruct((n,), jnp.float32),
            jax.ShapeDtypeStruct((n,), jnp.float32),
            jax.ShapeDtypeStruct((_NW, _LANES), jnp.float32),
        ),
        mesh=mesh,
        scratch_types=[
            pltpu.VMEM((chunk * _N_EXPERTS,), jnp.float32),
            pltpu.VMEM((chunk,), jnp.int32),
            pltpu.VMEM((chunk,), jnp.int32),
            pltpu.VMEM((chunk,), jnp.float32),
            pltpu.VMEM((chunk,), jnp.float32),
            pltpu.VMEM((_LANES,), jnp.float32),
        ],
    )
    def sc_router(lg_hbm, i1_hbm, i2_hbm, p1_hbm, p2_hbm, pc_hbm,
                  lg_v, i1_v, i2_v, p1_v, p2_v, cnt_v):
        wid = lax.axis_index("s") * 2 + lax.axis_index("c")
        base = wid * chunk
        for e in range(_N_EXPERTS):
            pltpu.sync_copy(lg_hbm.at[pl.ds(e * n + base, chunk)],
                            lg_v.at[pl.ds(e * chunk, chunk)])
        iota = lax.iota(jnp.int32, _L)
        zero16 = jnp.zeros((_L,), jnp.float32)
        neg_big = jnp.float32(-3.0e38)

        def body(g, accs):
            off = g * _L
            ls = [lg_v[pl.ds(e * chunk + off, _L)]
                  for e in range(_N_EXPERTS)]
            m = ls[0]
            for e in range(1, _N_EXPERTS):
                m = jnp.maximum(m, ls[e])
            i1 = jnp.full((_L,), _N_EXPERTS, jnp.int32)
            for e in range(_N_EXPERTS - 1, -1, -1):
                i1 = jnp.where(ls[e] == m, jnp.int32(e), i1)
            v2 = jnp.full((_L,), neg_big, jnp.float32)
            for e in range(_N_EXPERTS):
                v2 = jnp.maximum(v2, jnp.where(i1 == e, neg_big, ls[e]))
            i2 = jnp.full((_L,), _N_EXPERTS, jnp.int32)
            for e in range(_N_EXPERTS - 1, -1, -1):
                hit2 = jnp.logical_and(ls[e] == v2, i1 != e)
                i2 = jnp.where(hit2, jnp.int32(e), i2)
            s = zero16
            for e in range(_N_EXPERTS):
                s = s + jnp.exp(ls[e] - m)
            i1_v[pl.ds(off, _L)] = i1
            i2_v[pl.ds(off, _L)] = i2
            p1_v[pl.ds(off, _L)] = 1.0 / s
            p2_v[pl.ds(off, _L)] = jnp.exp(v2 - m) / s
            new_accs = []
            for e in range(_N_EXPERTS):
                hit = jnp.logical_or(i1 == e, i2 == e)
                new_accs.append(accs[e] + jnp.where(hit, 1.0, 0.0))
            return tuple(new_accs)

        accs = lax.fori_loop(0, groups, body,
                             tuple(zero16 for _ in range(_N_EXPERTS)))
        for e in range(_N_EXPERTS):
            cnt_v[pl.ds(e * _L, _L)] = accs[e]
        pltpu.sync_copy(i1_v, i1_hbm.at[pl.ds(base, chunk)])
        pltpu.sync_copy(i2_v, i2_hbm.at[pl.ds(base, chunk)])
        pltpu.sync_copy(p1_v, p1_hbm.at[pl.ds(base, chunk)])
        pltpu.sync_copy(p2_v, p2_hbm.at[pl.ds(base, chunk)])
        pltpu.sync_copy(cnt_v, pc_hbm.at[wid])

    return sc_router(logits_flat)


def kernel(hidden_states, gate_weight):
    n, d = hidden_states.shape
    tile = _TILE
    wpad = jnp.pad(gate_weight, ((0, _LANES - _N_EXPERTS), (0, 0)))
    logits_t = pl.pallas_call(
        _logits_body,
        grid=(n // tile,),
        in_specs=[
            pl.BlockSpec((tile, d), lambda i: (i, 0)),
            pl.BlockSpec((_LANES, d), lambda i: (0, 0)),
        ],
        out_specs=pl.BlockSpec((_N_EXPERTS, tile), lambda i: (0, i)),
        out_shape=jax.ShapeDtypeStruct((_N_EXPERTS, n), jnp.float32),
    )(hidden_states, wpad)
    i1, i2, p1, p2, partials = _sc_router(logits_t.reshape(-1), n)
    aux, loadp = pl.pallas_call(
        functools.partial(_stats_body, n_total=n),
        out_shape=(
            jax.ShapeDtypeStruct((1, _LANES), jnp.float32),
            jax.ShapeDtypeStruct((1, _LANES), jnp.float32),
        ),
    )(partials)
    idx = jnp.stack([i1, i2], axis=1)
    prob = jnp.stack([p1, p2], axis=1)
    return (idx, prob, aux[0, 0], loadp[0, :_N_EXPERTS])
